# Initial kernel scaffold; baseline (speedup 1.0000x reference)
import jax
import jax.numpy as jnp
from jax.experimental import pallas as pl


def kernel(input_ids, weight, hash_a, hash_b, sign_a, sign_b):
    raise NotImplementedError("write your pallas kernel here")


# speedup vs baseline: 4.6199x; 4.6199x over previous
import functools

import jax
import jax.numpy as jnp
from jax import lax
from jax.experimental import pallas as pl
from jax.experimental.pallas import tpu as pltpu
from jax.experimental.pallas import tpu_sc as plsc

_BUCKET = 100000
_D = 64
_NC = 2
_NS = 16
_NW = _NC * _NS
_L = 16

_CHUNK = 128


def _scale_body(w_ref, o_ref):
    s = jnp.where(pl.program_id(0) == 0, -0.5, 0.5).astype(jnp.float32)
    o_ref[...] = (w_ref[...] * s)[None]


_SCALE_BLK = 5000


def _build_aug(weight):
    out = pl.pallas_call(
        _scale_body,
        grid=(2, _BUCKET // _SCALE_BLK),
        in_specs=[pl.BlockSpec((_SCALE_BLK, _D), lambda i, j: (j, 0))],
        out_specs=pl.BlockSpec((1, _SCALE_BLK, _D), lambda i, j: (i, j, 0)),
        out_shape=jax.ShapeDtypeStruct((2, _BUCKET, _D), jnp.float32),
    )(weight)
    return out.reshape(2 * _BUCKET, _D)


def _make_sc_call(n_total):
    assert n_total % (_NW * _CHUNK) == 0
    n_per_w = n_total // _NW
    n_chunks = n_per_w // _CHUNK
    mesh = plsc.VectorSubcoreMesh(core_axis_name="c", subcore_axis_name="s")

    @functools.partial(
        pl.kernel,
        mesh=mesh,
        compiler_params=pltpu.CompilerParams(use_tc_tiling_on_sc=False),
        out_type=jax.ShapeDtypeStruct((n_total, _D), jnp.float32),
        scratch_types=[
            pltpu.VMEM((_L,), jnp.int32),
            pltpu.VMEM((_CHUNK,), jnp.int32),
            pltpu.VMEM((_CHUNK,), jnp.int32),
            pltpu.VMEM((_CHUNK,), jnp.int32),
            pltpu.VMEM((_CHUNK, _D), jnp.float32),
            pltpu.VMEM((_CHUNK, _D), jnp.float32),
            pltpu.SemaphoreType.DMA,
            pltpu.SemaphoreType.DMA,
        ],
    )
    def sc_call(aug_hbm, ids_hbm, hp_hbm, out_hbm,
                hp_v, ids_v, idx0_v, idx1_v, r0_v, r1_v, sem0, sem1):
        wid = lax.axis_index("s") * _NC + lax.axis_index("c")
        base = wid * n_per_w
        pltpu.sync_copy(hp_hbm, hp_v)
        hpv = hp_v[...]
        ha0, ha1 = hpv[0], hpv[1]
        hb0, hb1 = hpv[2], hpv[3]
        sa0, sa1 = hpv[4], hpv[5]
        sb0, sb1 = hpv[6], hpv[7]

        def chunk_body(ci, carry):
            off = base + ci * _CHUNK
            pltpu.sync_copy(ids_hbm.at[pl.ds(off, _CHUNK)], ids_v)
            for g in range(_CHUNK // _L):
                v = ids_v[pl.ds(g * _L, _L)]
                b0 = jnp.mod(v * ha0 + hb0, _BUCKET)
                m0 = (v * sa0 + sb0) & 1
                idx0_v[pl.ds(g * _L, _L)] = b0 + m0 * _BUCKET
                b1 = jnp.mod(v * ha1 + hb1, _BUCKET)
                m1 = (v * sa1 + sb1) & 1
                idx1_v[pl.ds(g * _L, _L)] = b1 + m1 * _BUCKET
            cp0 = pltpu.async_copy(aug_hbm.at[idx0_v], r0_v, sem0)
            cp1 = pltpu.async_copy(aug_hbm.at[idx1_v], r1_v, sem1)
            cp0.wait()
            cp1.wait()

            def add_body(i, c):
                for j in range(_D // _L):
                    sl = pl.ds(j * _L, _L)
                    r0_v[i, sl] = r0_v[i, sl] + r1_v[i, sl]
                return c

            lax.fori_loop(0, _CHUNK, add_body, 0)
            pltpu.sync_copy(r0_v, out_hbm.at[pl.ds(off, _CHUNK)])
            return carry

        lax.fori_loop(0, n_chunks, chunk_body, 0)

    return sc_call


def kernel(input_ids, weight, hash_a, hash_b, sign_a, sign_b):
    batch, fields = input_ids.shape
    n_total = batch * fields
    aug = _build_aug(weight)
    ids_flat = input_ids.reshape(n_total)
    hp = jnp.concatenate(
        [hash_a, hash_b, sign_a, sign_b,
         jnp.zeros((_L - 8,), jnp.int32)]).astype(jnp.int32)
    out = _make_sc_call(n_total)(aug, ids_flat, hp)
    return out.reshape(batch, fields, _D)


# speedup vs baseline: 5.1268x; 1.1097x over previous
import functools

import jax
import jax.numpy as jnp
from jax import lax
from jax.experimental import pallas as pl
from jax.experimental.pallas import tpu as pltpu
from jax.experimental.pallas import tpu_sc as plsc

_BUCKET = 100000
_D = 64
_NC = 2
_NS = 16
_NW = _NC * _NS
_L = 16

_CHUNK = 128


def _scale_body(w_ref, o_ref):
    s = jnp.where(pl.program_id(0) == 0, -0.5, 0.5).astype(jnp.float32)
    o_ref[...] = (w_ref[...] * s)[None]


_SCALE_BLK = 5000


def _build_aug(weight):
    out = pl.pallas_call(
        _scale_body,
        grid=(2, _BUCKET // _SCALE_BLK),
        in_specs=[pl.BlockSpec((_SCALE_BLK, _D), lambda i, j: (j, 0))],
        out_specs=pl.BlockSpec((1, _SCALE_BLK, _D), lambda i, j: (i, j, 0)),
        out_shape=jax.ShapeDtypeStruct((2, _BUCKET, _D), jnp.float32),
    )(weight)
    return out.reshape(2 * _BUCKET, _D)


def _make_sc_call(n_total):
    assert n_total % (_NW * _CHUNK) == 0
    n_per_w = n_total // _NW
    n_chunks = n_per_w // _CHUNK
    mesh = plsc.VectorSubcoreMesh(core_axis_name="c", subcore_axis_name="s")

    @functools.partial(
        pl.kernel,
        mesh=mesh,
        compiler_params=pltpu.CompilerParams(use_tc_tiling_on_sc=False),
        out_type=jax.ShapeDtypeStruct((n_total, _D), jnp.float32),
        scratch_types=[
            pltpu.VMEM((_L,), jnp.int32),
            pltpu.VMEM((_CHUNK,), jnp.int32),
            pltpu.VMEM((_CHUNK,), jnp.int32),
            pltpu.VMEM((_CHUNK,), jnp.int32),
            pltpu.VMEM((_CHUNK, _D), jnp.float32),
            pltpu.VMEM((_CHUNK, _D), jnp.float32),
            pltpu.SemaphoreType.DMA,
            pltpu.SemaphoreType.DMA,
        ],
    )
    def sc_call(aug_hbm, ids_hbm, hp_hbm, out_hbm,
                hp_v, ids_v, idx0_v, idx1_v, r0_v, r1_v, sem0, sem1):
        wid = lax.axis_index("s") * _NC + lax.axis_index("c")
        base = wid * n_per_w
        pltpu.sync_copy(hp_hbm, hp_v)
        hpv = hp_v[...]
        ha0, ha1 = hpv[0], hpv[1]
        hb0, hb1 = hpv[2], hpv[3]
        sa0, sa1 = hpv[4], hpv[5]
        sb0, sb1 = hpv[6], hpv[7]

        def chunk_body(ci, carry):
            off = base + ci * _CHUNK
            pltpu.sync_copy(ids_hbm.at[pl.ds(off, _CHUNK)], ids_v)
            for g in range(_CHUNK // _L):
                v = ids_v[pl.ds(g * _L, _L)]
                b0 = jnp.mod(v * ha0 + hb0, _BUCKET)
                m0 = (v * sa0 + sb0) & 1
                idx0_v[pl.ds(g * _L, _L)] = b0 + m0 * _BUCKET
                b1 = jnp.mod(v * ha1 + hb1, _BUCKET)
                m1 = (v * sa1 + sb1) & 1
                idx1_v[pl.ds(g * _L, _L)] = b1 + m1 * _BUCKET
            cp0 = pltpu.async_copy(aug_hbm.at[idx0_v], r0_v, sem0)
            cp0.wait()
            cp1 = pltpu.async_copy(aug_hbm.at[idx1_v], r0_v, sem1, add=True)
            cp1.wait()
            pltpu.sync_copy(r0_v, out_hbm.at[pl.ds(off, _CHUNK)])
            return carry

        lax.fori_loop(0, n_chunks, chunk_body, 0)

    return sc_call


def kernel(input_ids, weight, hash_a, hash_b, sign_a, sign_b):
    batch, fields = input_ids.shape
    n_total = batch * fields
    aug = _build_aug(weight)
    ids_flat = input_ids.reshape(n_total)
    hp = jnp.concatenate(
        [hash_a, hash_b, sign_a, sign_b,
         jnp.zeros((_L - 8,), jnp.int32)]).astype(jnp.int32)
    out = _make_sc_call(n_total)(aug, ids_flat, hp)
    return out.reshape(batch, fields, _D)


# speedup vs baseline: 6.0826x; 1.1864x over previous
import functools

import jax
import jax.numpy as jnp
from jax import lax
from jax.experimental import pallas as pl
from jax.experimental.pallas import tpu as pltpu
from jax.experimental.pallas import tpu_sc as plsc

_BUCKET = 100000
_D = 64
_NC = 2
_NS = 16
_NW = _NC * _NS
_L = 16

_CHUNK = 128


def _scale_body(w_ref, o_ref):
    s = jnp.where(pl.program_id(0) == 0, -0.5, 0.5).astype(jnp.float32)
    o_ref[...] = (w_ref[...] * s)[None]


_SCALE_BLK = 5000


def _build_aug(weight):
    out = pl.pallas_call(
        _scale_body,
        grid=(2, _BUCKET // _SCALE_BLK),
        in_specs=[pl.BlockSpec((_SCALE_BLK, _D), lambda i, j: (j, 0))],
        out_specs=pl.BlockSpec((1, _SCALE_BLK, _D), lambda i, j: (i, j, 0)),
        out_shape=jax.ShapeDtypeStruct((2, _BUCKET, _D), jnp.float32),
    )(weight)
    return out.reshape(2 * _BUCKET, _D)


_NSLOT = 4


def _make_sc_call(n_total):
    assert n_total % (_NW * _CHUNK * _NSLOT) == 0
    n_per_w = n_total // _NW
    n_chunks = n_per_w // _CHUNK
    n_blocks = n_chunks // _NSLOT
    mesh = plsc.VectorSubcoreMesh(core_axis_name="c", subcore_axis_name="s")

    scratch = (
        [pltpu.VMEM((_L,), jnp.int32)]
        + [pltpu.VMEM((_CHUNK,), jnp.int32) for _ in range(_NSLOT)]
        + [pltpu.VMEM((_CHUNK,), jnp.int32) for _ in range(2 * _NSLOT)]
        + [pltpu.VMEM((_CHUNK, _D), jnp.float32) for _ in range(_NSLOT)]
        + [pltpu.SemaphoreType.DMA for _ in range(4 * _NSLOT)]
    )

    @functools.partial(
        pl.kernel,
        mesh=mesh,
        compiler_params=pltpu.CompilerParams(use_tc_tiling_on_sc=False),
        out_type=jax.ShapeDtypeStruct((n_total, _D), jnp.float32),
        scratch_types=scratch,
    )
    def sc_call(aug_hbm, ids_hbm, hp_hbm, out_hbm, hp_v, *bufs):
        ids_v = bufs[0:_NSLOT]
        idx0_v = bufs[_NSLOT:2 * _NSLOT]
        idx1_v = bufs[2 * _NSLOT:3 * _NSLOT]
        r_v = bufs[3 * _NSLOT:4 * _NSLOT]
        ids_s = bufs[4 * _NSLOT:5 * _NSLOT]
        g0_s = bufs[5 * _NSLOT:6 * _NSLOT]
        ga_s = bufs[6 * _NSLOT:7 * _NSLOT]
        out_s = bufs[7 * _NSLOT:8 * _NSLOT]

        wid = lax.axis_index("s") * _NC + lax.axis_index("c")
        base = wid * n_per_w
        pltpu.sync_copy(hp_hbm, hp_v)
        hpv = hp_v[...]
        ha0, ha1 = hpv[0], hpv[1]
        hb0, hb1 = hpv[2], hpv[3]
        sa0, sa1 = hpv[4], hpv[5]
        sb0, sb1 = hpv[6], hpv[7]

        def ids_start(c, k):
            pltpu.async_copy(
                ids_hbm.at[pl.ds(base + c * _CHUNK, _CHUNK)], ids_v[k],
                ids_s[k])

        def gather0_start(k):
            pltpu.async_copy(aug_hbm.at[idx0_v[k]], r_v[k], g0_s[k])

        def gadd_start(k):
            pltpu.async_copy(aug_hbm.at[idx1_v[k]], r_v[k], ga_s[k],
                             add=True)

        def out_start(c, k):
            pltpu.async_copy(
                r_v[k], out_hbm.at[pl.ds(base + c * _CHUNK, _CHUNK)],
                out_s[k])

        def gather0_wait(k):
            pltpu.make_async_copy(aug_hbm.at[idx0_v[k]], r_v[k],
                                  g0_s[k]).wait()

        def gadd_wait(k):
            pltpu.make_async_copy(aug_hbm.at[idx1_v[k]], r_v[k],
                                  ga_s[k]).wait()

        def out_wait(c, k):
            pltpu.make_async_copy(
                r_v[k], out_hbm.at[pl.ds(base + c * _CHUNK, _CHUNK)],
                out_s[k]).wait()

        def compute_idx(k):
            for g in range(_CHUNK // _L):
                v = ids_v[k][pl.ds(g * _L, _L)]
                b0 = jnp.mod(v * ha0 + hb0, _BUCKET)
                m0 = (v * sa0 + sb0) & 1
                idx0_v[k][pl.ds(g * _L, _L)] = b0 + m0 * _BUCKET
                b1 = jnp.mod(v * ha1 + hb1, _BUCKET)
                m1 = (v * sa1 + sb1) & 1
                idx1_v[k][pl.ds(g * _L, _L)] = b1 + m1 * _BUCKET

        for k in range(_NSLOT):
            ids_start(k, k)

        def block_body(b, carry):
            for k in range(_NSLOT):
                c = b * _NSLOT + k
                pltpu.make_async_copy(
                    ids_hbm.at[pl.ds(base + c * _CHUNK, _CHUNK)], ids_v[k],
                    ids_s[k]).wait()
                compute_idx(k)

                @pl.when(b < n_blocks - 1)
                def _():
                    ids_start(c + _NSLOT, k)

                @pl.when(b >= 1)
                def _():
                    out_wait(c - _NSLOT, k)

                gather0_start(k)
                k1 = (k - 1) % _NSLOT
                if k == 0:
                    @pl.when(b >= 1)
                    def _():
                        gather0_wait(k1)
                        gadd_start(k1)
                else:
                    gather0_wait(k1)
                    gadd_start(k1)
                k2 = (k - 2) % _NSLOT
                c2 = c - 2
                if k in (0, 1):
                    @pl.when(b >= 1)
                    def _():
                        gadd_wait(k2)
                        out_start(c2, k2)
                else:
                    gadd_wait(k2)
                    out_start(c2, k2)
            return carry

        lax.fori_loop(0, n_blocks, block_body, 0)

        n = n_chunks
        gather0_wait(_NSLOT - 1)
        gadd_start(_NSLOT - 1)
        gadd_wait(_NSLOT - 2)
        out_start(n - 2, _NSLOT - 2)
        gadd_wait(_NSLOT - 1)
        out_start(n - 1, _NSLOT - 1)
        for k in range(_NSLOT):
            out_wait(n - _NSLOT + k, k)

    return sc_call


def kernel(input_ids, weight, hash_a, hash_b, sign_a, sign_b):
    batch, fields = input_ids.shape
    n_total = batch * fields
    aug = _build_aug(weight)
    ids_flat = input_ids.reshape(n_total)
    hp = jnp.concatenate(
        [hash_a, hash_b, sign_a, sign_b,
         jnp.zeros((_L - 8,), jnp.int32)]).astype(jnp.int32)
    out = _make_sc_call(n_total)(aug, ids_flat, hp)
    return out.reshape(batch, fields, _D)
